# baseline (speedup 1.0000x reference)
import functools

import jax
import jax.numpy as jnp
from jax import lax
from jax.experimental import pallas as pl
from jax.experimental.pallas import tpu as pltpu
from jax.experimental.pallas import tpu_sc as plsc

_NC, _NS, _L = 2, 16, 16
_NW = _NC * _NS


def kernel(x, pos_table):
    B, T, D = x.shape
    n = B * T * D
    t_per_w = T // _NW
    chunk_rows = 32
    chunk = chunk_rows * D
    n_chunks = t_per_w // chunk_rows
    n_steps = n_chunks * B

    xf = x.reshape(n)
    pf = pos_table.reshape(T * D)

    @functools.partial(
        pl.kernel,
        out_type=jax.ShapeDtypeStruct((n,), jnp.float32),
        mesh=plsc.VectorSubcoreMesh(core_axis_name="c", subcore_axis_name="s"),
        scratch_types=[
            pltpu.VMEM((2, chunk), jnp.float32),
            pltpu.VMEM((2, chunk), jnp.float32),
            pltpu.SemaphoreType.DMA,
            pltpu.SemaphoreType.DMA,
            pltpu.SemaphoreType.DMA,
            pltpu.SemaphoreType.DMA,
            pltpu.SemaphoreType.DMA,
            pltpu.SemaphoreType.DMA,
        ],
    )
    def sc_add(x_hbm, pos_hbm, out_hbm, xbuf, pbuf, xi0, xi1, xo0, xo1, pi0, pi1):
        wid = lax.axis_index("s") * _NC + lax.axis_index("c")
        base = wid * (t_per_w * D)
        xin_sems = (xi0, xi1)
        xout_sems = (xo0, xo1)
        pin_sems = (pi0, pi1)

        def x_off(k):
            c, b = divmod(k, B)
            return b * (T * D) + base + c * chunk

        loads = {}
        stores = {}
        ploads = {}
        loads[0] = pltpu.async_copy(
            x_hbm.at[pl.ds(x_off(0), chunk)], xbuf.at[0], xin_sems[0])
        ploads[0] = pltpu.async_copy(
            pos_hbm.at[pl.ds(base, chunk)], pbuf.at[0], pin_sems[0])

        for k in range(n_steps):
            cur = k % 2
            c = k // B
            if k % B == 0:
                ploads[c].wait()
                if c + 1 < n_chunks:
                    nxt_p = (c + 1) % 2
                    ploads[c + 1] = pltpu.async_copy(
                        pos_hbm.at[pl.ds(base + (c + 1) * chunk, chunk)],
                        pbuf.at[nxt_p], pin_sems[nxt_p])
            loads[k].wait()
            if k + 1 < n_steps:
                nxt = (k + 1) % 2
                if k - 1 >= 0:
                    stores[k - 1].wait()
                loads[k + 1] = pltpu.async_copy(
                    x_hbm.at[pl.ds(x_off(k + 1), chunk)], xbuf.at[nxt],
                    xin_sems[nxt])

            xcur = xbuf.at[cur]
            pcur = pbuf.at[c % 2]

            @plsc.parallel_loop(0, chunk // _L, unroll=16)
            def _(i, xc=xcur, pc=pcur):
                sl = pl.ds(i * _L, _L)
                xc[sl] = xc[sl] + pc[sl]
            stores[k] = pltpu.async_copy(
                xcur, out_hbm.at[pl.ds(x_off(k), chunk)], xout_sems[cur])
        stores[n_steps - 2].wait()
        stores[n_steps - 1].wait()

    out = sc_add(xf, pf)
    return out.reshape(B, T, D)


# speedup vs baseline: 5.1359x; 5.1359x over previous
import jax
import jax.numpy as jnp
from jax.experimental import pallas as pl
from jax.experimental.pallas import tpu as pltpu


_ROWS_PER_BLOCK = 2048


def _add_kernel(x_ref, pos_ref, o_ref):
    o_ref[...] = x_ref[...] + pos_ref[...]


def kernel(x, pos_table):
    B, T, D = x.shape
    x2 = x.reshape(B * T, D)
    rb = _ROWS_PER_BLOCK
    blocks_per_batch = T // rb

    out = pl.pallas_call(
        _add_kernel,
        grid=(blocks_per_batch, B),
        in_specs=[
            pl.BlockSpec((rb, D), lambda i, b: (b * blocks_per_batch + i, 0)),
            pl.BlockSpec((rb, D), lambda i, b: (i, 0)),
        ],
        out_specs=pl.BlockSpec((rb, D), lambda i, b: (b * blocks_per_batch + i, 0)),
        out_shape=jax.ShapeDtypeStruct((B * T, D), x.dtype),
        compiler_params=pltpu.CompilerParams(
            dimension_semantics=("parallel", "arbitrary"),
        ),
    )(x2, pos_table)
    return out.reshape(B, T, D)
